# Initial kernel scaffold; baseline (speedup 1.0000x reference)
import jax
import jax.numpy as jnp
from jax.experimental import pallas as pl


def kernel(vocab_id_list, table):
    raise NotImplementedError("write your pallas kernel here")


# speedup vs baseline: 1.1023x; 1.1023x over previous
import functools

import jax
import jax.numpy as jnp
from jax import lax
from jax.experimental import pallas as pl
from jax.experimental.pallas import tpu as pltpu
from jax.experimental.pallas import tpu_sc as plsc

NC = 2
NS = 16
NW = NC * NS


def _emb_body(n_chunks, chunk, b_per_w, table_hbm, idx_hbm, out_hbm,
              idx_v, rows_v, sems):
    wid = lax.axis_index("s") * NC + lax.axis_index("c")
    base = wid * b_per_w

    def body(i, carry):
        off = base + i * chunk
        pltpu.sync_copy(idx_hbm.at[pl.ds(off, chunk)], idx_v)
        pltpu.async_copy(table_hbm.at[idx_v], rows_v, sems).wait()
        pltpu.sync_copy(rows_v, out_hbm.at[pl.ds(off, chunk)])
        return carry

    lax.fori_loop(0, n_chunks, body, 0)


def kernel(vocab_id_list, table):
    batch, hist = vocab_id_list.shape
    vocab, d = table.shape
    b = batch * hist
    idx = vocab_id_list.reshape(b).astype(jnp.int32)

    b_per_w = b // NW
    chunk = 1600
    while b_per_w % chunk:
        chunk //= 2
    n_chunks = b_per_w // chunk

    mesh = plsc.VectorSubcoreMesh(core_axis_name="c", subcore_axis_name="s")
    out = pl.kernel(
        functools.partial(_emb_body, n_chunks, chunk, b_per_w),
        out_type=jax.ShapeDtypeStruct((b, d), jnp.float32),
        mesh=mesh,
        compiler_params=pltpu.CompilerParams(use_tc_tiling_on_sc=False),
        scratch_types=[
            pltpu.VMEM((chunk,), jnp.int32),
            pltpu.VMEM((chunk, d), jnp.float32),
            pltpu.SemaphoreType.DMA,
        ],
    )(table, idx)
    return out.reshape(batch, hist, d)


# speedup vs baseline: 1.1129x; 1.0096x over previous
import functools

import jax
import jax.numpy as jnp
from jax import lax
from jax.experimental import pallas as pl
from jax.experimental.pallas import tpu as pltpu
from jax.experimental.pallas import tpu_sc as plsc

NC = 2
NS = 16
NW = NC * NS
NBUF = 4


def _emb_body(n_chunks, chunk, b_per_w, table_hbm, idx_hbm, out_hbm,
              idx_v, rows_v, sem_g, sem_o):
    wid = lax.axis_index("s") * NC + lax.axis_index("c")
    base = wid * b_per_w

    def load_idx(i, j):
        pltpu.sync_copy(idx_hbm.at[pl.ds(base + i * chunk, chunk)],
                        idx_v.at[j])

    def start_gather(j):
        pltpu.make_async_copy(table_hbm.at[idx_v.at[j]], rows_v.at[j],
                              sem_g.at[j]).start()

    def wait_gather(j):
        pltpu.make_async_copy(table_hbm.at[idx_v.at[j]], rows_v.at[j],
                              sem_g.at[j]).wait()

    def start_store(i, j):
        pltpu.make_async_copy(rows_v.at[j],
                              out_hbm.at[pl.ds(base + i * chunk, chunk)],
                              sem_o.at[j]).start()

    def wait_store(i, j):
        pltpu.make_async_copy(rows_v.at[j],
                              out_hbm.at[pl.ds(base + i * chunk, chunk)],
                              sem_o.at[j]).wait()

    load_idx(0, 0)
    start_gather(0)
    for i in range(1, NBUF):
        load_idx(i, i)
        start_gather(i)
        wait_gather(i - 1)
        start_store(i - 1, i - 1)

    def outer(g, carry):
        i0 = NBUF + g * NBUF
        for j in range(NBUF):
            i = i0 + j
            wait_store(i - NBUF, j)
            load_idx(i, j)
            start_gather(j)
            jp = (j - 1) % NBUF
            wait_gather(jp)
            start_store(i - 1, jp)
        return carry

    lax.fori_loop(0, (n_chunks - NBUF) // NBUF, outer, 0)

    last = n_chunks - 1
    jl = last % NBUF
    wait_gather(jl)
    start_store(last, jl)
    for k in range(NBUF):
        i = n_chunks - NBUF + k
        wait_store(i, i % NBUF)


def kernel(vocab_id_list, table):
    batch, hist = vocab_id_list.shape
    vocab, d = table.shape
    b = batch * hist
    idx = vocab_id_list.reshape(b).astype(jnp.int32)

    b_per_w = b // NW
    chunk = 800
    while b_per_w % (chunk * NBUF):
        chunk //= 2
    n_chunks = b_per_w // chunk

    mesh = plsc.VectorSubcoreMesh(core_axis_name="c", subcore_axis_name="s")
    out = pl.kernel(
        functools.partial(_emb_body, n_chunks, chunk, b_per_w),
        out_type=jax.ShapeDtypeStruct((b, d), jnp.float32),
        mesh=mesh,
        compiler_params=pltpu.CompilerParams(use_tc_tiling_on_sc=False),
        scratch_types=[
            pltpu.VMEM((NBUF, chunk), jnp.int32),
            pltpu.VMEM((NBUF, chunk, d), jnp.float32),
            pltpu.SemaphoreType.DMA((NBUF,)),
            pltpu.SemaphoreType.DMA((NBUF,)),
        ],
    )(table, idx)
    return out.reshape(batch, hist, d)
